# baseline (speedup 1.0000x reference)
import functools

import jax
import jax.numpy as jnp
from jax.experimental import pallas as pl
from jax.experimental.pallas import tpu as pltpu

B = 512


def _w_block(pos_c, sq_col, geomT_ref, r2_val, a, b):
    sq_row = geomT_ref[2:3, pl.ds(b * B, B)]
    crossm2 = jnp.dot(pos_c, geomT_ref[0:2, pl.ds(b * B, B)],
                      preferred_element_type=jnp.float32)
    d2 = jnp.maximum((sq_col + sq_row) + crossm2, 1e-12)
    row_ids = a * B + jax.lax.broadcasted_iota(jnp.int32, (B, B), 0)
    col_ids = b * B + jax.lax.broadcasted_iota(jnp.int32, (B, B), 1)
    mask = (d2 <= r2_val) & (row_ids != col_ids)
    return jnp.where(mask, jax.lax.rsqrt(d2), 0.0)


def _sq_col(pos_c):
    px = pos_c[:, 0:1]
    py = pos_c[:, 1:2]
    return px * px + py * py


def _window(s_ref, nb, a):
    return s_ref[a], s_ref[nb + a]


def _deg_kernel(nb, s_ref, geomT_ref, pos_c_ref, x_ref, W1_ref, r2_ref,
                dinv_ref, M1_ref):
    a = pl.program_id(0)
    r2 = r2_ref[0:1, 0:1]
    pos_c = pos_c_ref[...]
    sq_col = _sq_col(pos_c)
    lo, hi = _window(s_ref, nb, a)

    def _blk(b, y):
        w = _w_block(pos_c, sq_col, geomT_ref, r2, a, b)
        return y + jnp.sum(w, axis=1, keepdims=True)

    acc = jax.lax.fori_loop(lo, hi + 1, _blk, jnp.zeros((B, 1), jnp.float32))
    deg = acc + 1.0
    dinv = jax.lax.rsqrt(deg)
    dinv_ref[...] = jnp.broadcast_to(dinv, (B, 8))
    xw = jnp.dot(x_ref[...], W1_ref[...], preferred_element_type=jnp.float32)
    M1_ref[...] = dinv * xw


def _agg_kernel(nb, relu_next, s_ref, geomT_ref, pos_c_ref, M_ref, dinv_ref,
                Wn_ref, bias_ref, r2_ref, out_ref):
    a = pl.program_id(0)
    r2 = r2_ref[0:1, 0:1]
    pos_c = pos_c_ref[...]
    sq_col = _sq_col(pos_c)
    lo, hi = _window(s_ref, nb, a)

    def _blk(b, y):
        w = _w_block(pos_c, sq_col, geomT_ref, r2, a, b)
        return y + jnp.dot(w, M_ref[pl.ds(b * B, B), :],
                           preferred_element_type=jnp.float32)

    acc = jax.lax.fori_loop(lo, hi + 1, _blk, M_ref[pl.ds(a * B, B), :])
    dinv = dinv_ref[:, 0:1]
    y = dinv * acc + bias_ref[0:1, :]
    if relu_next:
        h = jax.nn.relu(y)
        out_ref[...] = dinv * jnp.dot(h, Wn_ref[...],
                                      preferred_element_type=jnp.float32)
    else:
        out_ref[...] = y


def kernel(x, pos, r, W1, b1, W2, b2):
    n, feat = x.shape
    h1 = W1.shape[1]
    h2 = W2.shape[1]
    nb = -(-n // B)
    np_ = nb * B

    perm = jnp.argsort(pos[:, 1])
    pos_s = pos[perm]
    x_s = x[perm]

    pad = np_ - n
    fill = 1e5 + 1e3 * jnp.arange(pad, dtype=jnp.float32)
    pos_p = jnp.concatenate([pos_s, jnp.stack([fill, fill], axis=1)], axis=0)
    x_p = jnp.concatenate([x_s, jnp.zeros((pad, feat), x.dtype)], axis=0)
    sq_p = jnp.sum(pos_p * pos_p, axis=1)
    geomT = jnp.concatenate([-2.0 * pos_p.T, sq_p[None, :],
                             jnp.zeros((5, np_), jnp.float32)], axis=0)
    r_f = jnp.asarray(r, jnp.float32)
    r2_b = jnp.full((1, 128), r_f * r_f, jnp.float32)
    b1_2 = b1.reshape(1, h1)
    b2_2 = b2.reshape(1, h2)

    yb = pos_p[:, 1].reshape(nb, B)
    xb = pos_p[:, 0].reshape(nb, B)
    ymin = jnp.min(yb, axis=1)
    ymax = jnp.max(yb, axis=1)
    mx = jnp.max(jnp.abs(xb), axis=1)
    my = jnp.max(jnp.abs(yb), axis=1)
    err = ((mx[:, None] * mx[None, :] + my[:, None] * my[None, :])
           * (1.2 / 64.0) + 0.01)
    reach = jnp.sqrt(r_f * r_f + err)
    act = ((ymin[None, :] <= ymax[:, None] + reach)
           & (ymax[None, :] >= ymin[:, None] - reach))
    idx = jnp.arange(nb, dtype=jnp.int32)
    lo = jnp.min(jnp.where(act, idx[None, :], nb), axis=1)
    hi = jnp.max(jnp.where(act, idx[None, :], -1), axis=1)
    scal = jnp.concatenate([lo, hi])

    full = lambda shape: pl.BlockSpec(shape, lambda a, s: (0, 0))
    rowblk = lambda w: pl.BlockSpec((B, w), lambda a, s: (a, 0))

    dinv, M1 = pl.pallas_call(
        functools.partial(_deg_kernel, nb),
        grid_spec=pltpu.PrefetchScalarGridSpec(
            num_scalar_prefetch=1,
            grid=(nb,),
            in_specs=[full((8, np_)), rowblk(2), rowblk(feat),
                      full((feat, h1)), full((1, 128))],
            out_specs=[rowblk(8), rowblk(h1)],
        ),
        out_shape=[jax.ShapeDtypeStruct((np_, 8), jnp.float32),
                   jax.ShapeDtypeStruct((np_, h1), jnp.float32)],
    )(scal, geomT, pos_p, x_p, W1, r2_b)

    M2 = pl.pallas_call(
        functools.partial(_agg_kernel, nb, True),
        grid_spec=pltpu.PrefetchScalarGridSpec(
            num_scalar_prefetch=1,
            grid=(nb,),
            in_specs=[full((8, np_)), rowblk(2), full((np_, h1)), rowblk(8),
                      full((h1, h2)), full((1, h1)), full((1, 128))],
            out_specs=rowblk(h2),
        ),
        out_shape=jax.ShapeDtypeStruct((np_, h2), jnp.float32),
    )(scal, geomT, pos_p, M1, dinv, W2, b1_2, r2_b)

    out_s = pl.pallas_call(
        functools.partial(_agg_kernel, nb, False),
        grid_spec=pltpu.PrefetchScalarGridSpec(
            num_scalar_prefetch=1,
            grid=(nb,),
            in_specs=[full((8, np_)), rowblk(2), full((np_, h2)), rowblk(8),
                      full((h1, h2)), full((1, h2)), full((1, 128))],
            out_specs=rowblk(h2),
        ),
        out_shape=jax.ShapeDtypeStruct((np_, h2), jnp.float32),
    )(scal, geomT, pos_p, M2, dinv, W2, b2_2, r2_b)

    inv = jnp.zeros((n,), jnp.int32).at[perm].set(
        jnp.arange(n, dtype=jnp.int32))
    return out_s[:n][inv]


# speedup vs baseline: 1.5570x; 1.5570x over previous
import functools

import jax
import jax.numpy as jnp
from jax.experimental import pallas as pl
from jax.experimental.pallas import tpu as pltpu

B = 512


def _w_block(pos_c, sq_col, geomT_ref, r2_val, a, b):
    sq_row = geomT_ref[2:3, pl.ds(b * B, B)]
    crossm2 = jnp.dot(pos_c, geomT_ref[0:2, pl.ds(b * B, B)],
                      preferred_element_type=jnp.float32)
    d2 = jnp.maximum((sq_col + sq_row) + crossm2, 1e-12)
    row_ids = a * B + jax.lax.broadcasted_iota(jnp.int32, (B, B), 0)
    col_ids = b * B + jax.lax.broadcasted_iota(jnp.int32, (B, B), 1)
    mask = (d2 <= r2_val) & (row_ids != col_ids)
    return jnp.where(mask, jax.lax.rsqrt(d2), 0.0)


def _sq_col(pos_c):
    px = pos_c[:, 0:1]
    py = pos_c[:, 1:2]
    return px * px + py * py


def _window(s_ref, nb, a):
    return s_ref[a], s_ref[nb + a]


def _deg_kernel(nb, s_ref, geomT_ref, pos_c_ref, x_ref, W1_ref, r2_ref,
                dinv_ref, M1_ref):
    a = pl.program_id(0)
    r2 = r2_ref[0:1, 0:1]
    pos_c = pos_c_ref[...]
    sq_col = _sq_col(pos_c)
    lo, hi = _window(s_ref, nb, a)

    def _blk(b, y):
        w = _w_block(pos_c, sq_col, geomT_ref, r2, a, b)
        return y + jnp.sum(w, axis=1, keepdims=True)

    acc = jax.lax.fori_loop(lo, hi + 1, _blk, jnp.zeros((B, 1), jnp.float32))
    deg = acc + 1.0
    dinv = jax.lax.rsqrt(deg)
    dinv_ref[...] = jnp.broadcast_to(dinv, (B, 8))
    xw = jnp.dot(x_ref[...], W1_ref[...], preferred_element_type=jnp.float32)
    M1_ref[...] = dinv * xw


def _agg_kernel(nb, relu_next, s_ref, geomT_ref, pos_c_ref, M_ref, dinv_ref,
                Wn_ref, bias_ref, r2_ref, out_ref):
    a = pl.program_id(0)
    r2 = r2_ref[0:1, 0:1]
    pos_c = pos_c_ref[...]
    sq_col = _sq_col(pos_c)
    lo, hi = _window(s_ref, nb, a)

    def _blk(b, y):
        w = _w_block(pos_c, sq_col, geomT_ref, r2, a, b)
        return y + jnp.dot(w, M_ref[pl.ds(b * B, B), :],
                           preferred_element_type=jnp.float32)

    acc = jax.lax.fori_loop(lo, hi + 1, _blk, M_ref[pl.ds(a * B, B), :])
    dinv = dinv_ref[:, 0:1]
    y = dinv * acc + bias_ref[0:1, :]
    if relu_next:
        h = jax.nn.relu(y)
        out_ref[...] = dinv * jnp.dot(h, Wn_ref[...],
                                      preferred_element_type=jnp.float32)
    else:
        out_ref[...] = y


def kernel(x, pos, r, W1, b1, W2, b2):
    n, feat = x.shape
    h1 = W1.shape[1]
    h2 = W2.shape[1]
    nb = -(-n // B)
    np_ = nb * B

    perm = jnp.argsort(pos[:, 1])
    pos_s = pos[perm]
    x_s = x[perm]

    pad = np_ - n
    fill = 1e5 + 1e3 * jnp.arange(pad, dtype=jnp.float32)
    pos_p = jnp.concatenate([pos_s, jnp.stack([fill, fill], axis=1)], axis=0)
    x_p = jnp.concatenate([x_s, jnp.zeros((pad, feat), x.dtype)], axis=0)
    sq_p = jnp.sum(pos_p * pos_p, axis=1)
    geomT = jnp.concatenate([-2.0 * pos_p.T, sq_p[None, :],
                             jnp.zeros((5, np_), jnp.float32)], axis=0)
    r_f = jnp.asarray(r, jnp.float32)
    r2_b = jnp.full((1, 128), r_f * r_f, jnp.float32)
    b1_2 = b1.reshape(1, h1)
    b2_2 = b2.reshape(1, h2)

    valid = (jnp.arange(np_) < n).reshape(nb, B)
    yb = pos_p[:, 1].reshape(nb, B)
    xb = pos_p[:, 0].reshape(nb, B)
    ymin = jnp.min(jnp.where(valid, yb, 1e9), axis=1)
    ymax = jnp.max(jnp.where(valid, yb, -1e9), axis=1)
    mx = jnp.max(jnp.where(valid, jnp.abs(xb), 0.0), axis=1)
    my = jnp.max(jnp.where(valid, jnp.abs(yb), 0.0), axis=1)
    err = ((mx[:, None] * mx[None, :] + my[:, None] * my[None, :])
           * (1.2 / 64.0) + 0.01)
    reach = jnp.sqrt(r_f * r_f + err)
    act = ((ymin[None, :] <= ymax[:, None] + reach)
           & (ymax[None, :] >= ymin[:, None] - reach))
    idx = jnp.arange(nb, dtype=jnp.int32)
    lo = jnp.min(jnp.where(act, idx[None, :], nb), axis=1)
    hi = jnp.max(jnp.where(act, idx[None, :], -1), axis=1)
    scal = jnp.concatenate([lo, hi])

    full = lambda shape: pl.BlockSpec(shape, lambda a, s: (0, 0))
    rowblk = lambda w: pl.BlockSpec((B, w), lambda a, s: (a, 0))

    dinv, M1 = pl.pallas_call(
        functools.partial(_deg_kernel, nb),
        grid_spec=pltpu.PrefetchScalarGridSpec(
            num_scalar_prefetch=1,
            grid=(nb,),
            in_specs=[full((8, np_)), rowblk(2), rowblk(feat),
                      full((feat, h1)), full((1, 128))],
            out_specs=[rowblk(8), rowblk(h1)],
        ),
        out_shape=[jax.ShapeDtypeStruct((np_, 8), jnp.float32),
                   jax.ShapeDtypeStruct((np_, h1), jnp.float32)],
    )(scal, geomT, pos_p, x_p, W1, r2_b)

    M2 = pl.pallas_call(
        functools.partial(_agg_kernel, nb, True),
        grid_spec=pltpu.PrefetchScalarGridSpec(
            num_scalar_prefetch=1,
            grid=(nb,),
            in_specs=[full((8, np_)), rowblk(2), full((np_, h1)), rowblk(8),
                      full((h1, h2)), full((1, h1)), full((1, 128))],
            out_specs=rowblk(h2),
        ),
        out_shape=jax.ShapeDtypeStruct((np_, h2), jnp.float32),
    )(scal, geomT, pos_p, M1, dinv, W2, b1_2, r2_b)

    out_s = pl.pallas_call(
        functools.partial(_agg_kernel, nb, False),
        grid_spec=pltpu.PrefetchScalarGridSpec(
            num_scalar_prefetch=1,
            grid=(nb,),
            in_specs=[full((8, np_)), rowblk(2), full((np_, h2)), rowblk(8),
                      full((h1, h2)), full((1, h2)), full((1, 128))],
            out_specs=rowblk(h2),
        ),
        out_shape=jax.ShapeDtypeStruct((np_, h2), jnp.float32),
    )(scal, geomT, pos_p, M2, dinv, W2, b2_2, r2_b)

    inv = jnp.zeros((n,), jnp.int32).at[perm].set(
        jnp.arange(n, dtype=jnp.int32))
    return out_s[:n][inv]


# speedup vs baseline: 1.5746x; 1.0113x over previous
import functools

import jax
import jax.numpy as jnp
from jax.experimental import pallas as pl
from jax.experimental.pallas import tpu as pltpu

B = 512
KW = 11


def _w_math(pos_c, sq_col, sq_row, crossm2, r2_val, a, b):
    d2 = jnp.maximum((sq_col + sq_row) + crossm2, 1e-12)
    row_ids = a * B + jax.lax.broadcasted_iota(jnp.int32, (B, B), 0)
    col_ids = b * B + jax.lax.broadcasted_iota(jnp.int32, (B, B), 1)
    mask = (d2 <= r2_val) & (row_ids != col_ids)
    return jnp.where(mask, jax.lax.rsqrt(d2), 0.0)


def _w_from_blk(pos_c, sq_col, gblk, r2_val, a, b):
    crossm2 = jnp.dot(pos_c, gblk[0:2, :], preferred_element_type=jnp.float32)
    return _w_math(pos_c, sq_col, gblk[2:3, :], crossm2, r2_val, a, b)


def _w_dyn(pos_c, sq_col, geomT_ref, r2_val, a, b):
    gblk = geomT_ref[:, pl.ds(b * B, B)]
    return _w_from_blk(pos_c, sq_col, gblk, r2_val, a, b)


def _sq_col(pos_c):
    px = pos_c[:, 0:1]
    py = pos_c[:, 1:2]
    return px * px + py * py


def _deg_kernel(nb1, s_ref, *refs):
    gblks = refs[:KW]
    geomT_ref, pos_c_ref, x_ref, W1_ref, r2_ref, dinv_ref, M1_ref = refs[KW:]
    a = pl.program_id(0)
    lo = s_ref[a]
    hi = s_ref[nb1 + a]
    r2 = r2_ref[0:1, 0:1]
    pos_c = pos_c_ref[...]
    sq_col = _sq_col(pos_c)
    acc = jnp.zeros((B, 1), jnp.float32)
    for k in range(KW):
        b = jnp.minimum(lo + k, nb1 - 1)
        w = _w_from_blk(pos_c, sq_col, gblks[k][...], r2, a, b)
        acc = acc + jnp.sum(w, axis=1, keepdims=True)

    def _ov(b, y):
        w = _w_dyn(pos_c, sq_col, geomT_ref, r2, a, b)
        return y + jnp.sum(w, axis=1, keepdims=True)

    acc = jax.lax.fori_loop(lo + KW, hi + 1, _ov, acc)
    deg = acc + 1.0
    dinv = jax.lax.rsqrt(deg)
    dinv_ref[...] = jnp.broadcast_to(dinv, (B, 8))
    xw = jnp.dot(x_ref[...], W1_ref[...], preferred_element_type=jnp.float32)
    M1_ref[...] = dinv * xw


def _agg_kernel(nb1, relu_next, s_ref, *refs):
    gblks = refs[:KW]
    mblks = refs[KW:2 * KW]
    (geomT_ref, pos_c_ref, M_ref, ma_ref, dinv_ref, Wn_ref, bias_ref,
     r2_ref, out_ref) = refs[2 * KW:]
    a = pl.program_id(0)
    lo = s_ref[a]
    hi = s_ref[nb1 + a]
    r2 = r2_ref[0:1, 0:1]
    pos_c = pos_c_ref[...]
    sq_col = _sq_col(pos_c)
    y = ma_ref[...]
    for k in range(KW):
        b = jnp.minimum(lo + k, nb1 - 1)
        w = _w_from_blk(pos_c, sq_col, gblks[k][...], r2, a, b)
        y = y + jnp.dot(w, mblks[k][...], preferred_element_type=jnp.float32)

    def _ov(b, yy):
        w = _w_dyn(pos_c, sq_col, geomT_ref, r2, a, b)
        return yy + jnp.dot(w, M_ref[pl.ds(b * B, B), :],
                            preferred_element_type=jnp.float32)

    y = jax.lax.fori_loop(lo + KW, hi + 1, _ov, y)
    dinv = dinv_ref[:, 0:1]
    y = dinv * y + bias_ref[0:1, :]
    if relu_next:
        h = jax.nn.relu(y)
        out_ref[...] = dinv * jnp.dot(h, Wn_ref[...],
                                      preferred_element_type=jnp.float32)
    else:
        out_ref[...] = y


def kernel(x, pos, r, W1, b1, W2, b2):
    n, feat = x.shape
    h1 = W1.shape[1]
    h2 = W2.shape[1]
    nb = -(-n // B)
    nb1 = nb + 1
    np_ = nb1 * B

    perm = jnp.argsort(pos[:, 1])
    pos_s = pos[perm]
    x_s = x[perm]

    pad = np_ - n
    fill = 1e5 + 1e3 * jnp.arange(pad, dtype=jnp.float32)
    pos_p = jnp.concatenate([pos_s, jnp.stack([fill, fill], axis=1)], axis=0)
    x_p = jnp.concatenate([x_s, jnp.zeros((pad, feat), x.dtype)], axis=0)
    sq_p = jnp.sum(pos_p * pos_p, axis=1)
    geomT = jnp.concatenate([-2.0 * pos_p.T, sq_p[None, :],
                             jnp.zeros((5, np_), jnp.float32)], axis=0)
    r_f = jnp.asarray(r, jnp.float32)
    r2_b = jnp.full((1, 128), r_f * r_f, jnp.float32)
    b1_2 = b1.reshape(1, h1)
    b2_2 = b2.reshape(1, h2)

    valid = (jnp.arange(np_) < n).reshape(nb1, B)
    yb = pos_p[:, 1].reshape(nb1, B)
    xb = pos_p[:, 0].reshape(nb1, B)
    ymin = jnp.min(jnp.where(valid, yb, 1e9), axis=1)
    ymax = jnp.max(jnp.where(valid, yb, -1e9), axis=1)
    mx = jnp.max(jnp.where(valid, jnp.abs(xb), 0.0), axis=1)
    my = jnp.max(jnp.where(valid, jnp.abs(yb), 0.0), axis=1)
    err = ((mx[:, None] * mx[None, :] + my[:, None] * my[None, :])
           * (1.2 / 64.0) + 0.01)
    reach = jnp.sqrt(r_f * r_f + err)
    act = ((ymin[None, :] <= ymax[:, None] + reach)
           & (ymax[None, :] >= ymin[:, None] - reach))
    idx = jnp.arange(nb1, dtype=jnp.int32)
    lo = jnp.min(jnp.where(act, idx[None, :], nb1), axis=1)
    hi = jnp.max(jnp.where(act, idx[None, :], -1), axis=1)
    scal = jnp.concatenate([lo, hi])

    full = lambda shape: pl.BlockSpec(shape, lambda a, s: (0, 0))
    rowblk = lambda w: pl.BlockSpec((B, w), lambda a, s: (a, 0))

    def slotg(k):
        return pl.BlockSpec(
            (8, B), lambda a, s, k=k: (0, jnp.minimum(s[a] + k, nb1 - 1)))

    def slotm(k, w):
        return pl.BlockSpec(
            (B, w), lambda a, s, k=k: (jnp.minimum(s[a] + k, nb1 - 1), 0))

    gslots = [slotg(k) for k in range(KW)]

    dinv, M1 = pl.pallas_call(
        functools.partial(_deg_kernel, nb1),
        grid_spec=pltpu.PrefetchScalarGridSpec(
            num_scalar_prefetch=1,
            grid=(nb1,),
            in_specs=gslots + [full((8, np_)), rowblk(2), rowblk(feat),
                               full((feat, h1)), full((1, 128))],
            out_specs=[rowblk(8), rowblk(h1)],
        ),
        out_shape=[jax.ShapeDtypeStruct((np_, 8), jnp.float32),
                   jax.ShapeDtypeStruct((np_, h1), jnp.float32)],
    )(scal, *([geomT] * KW), geomT, pos_p, x_p, W1, r2_b)

    M2 = pl.pallas_call(
        functools.partial(_agg_kernel, nb1, True),
        grid_spec=pltpu.PrefetchScalarGridSpec(
            num_scalar_prefetch=1,
            grid=(nb1,),
            in_specs=(gslots + [slotm(k, h1) for k in range(KW)]
                      + [full((8, np_)), rowblk(2), full((np_, h1)),
                         rowblk(h1), rowblk(8), full((h1, h2)),
                         full((1, h1)), full((1, 128))]),
            out_specs=rowblk(h2),
        ),
        out_shape=jax.ShapeDtypeStruct((np_, h2), jnp.float32),
    )(scal, *([geomT] * KW), *([M1] * KW), geomT, pos_p, M1, M1, dinv,
      W2, b1_2, r2_b)

    out_s = pl.pallas_call(
        functools.partial(_agg_kernel, nb1, False),
        grid_spec=pltpu.PrefetchScalarGridSpec(
            num_scalar_prefetch=1,
            grid=(nb1,),
            in_specs=(gslots + [slotm(k, h2) for k in range(KW)]
                      + [full((8, np_)), rowblk(2), full((np_, h2)),
                         rowblk(h2), rowblk(8), full((h1, h2)),
                         full((1, h2)), full((1, 128))]),
            out_specs=rowblk(h2),
        ),
        out_shape=jax.ShapeDtypeStruct((np_, h2), jnp.float32),
    )(scal, *([geomT] * KW), *([M2] * KW), geomT, pos_p, M2, M2, dinv,
      W2, b2_2, r2_b)

    inv = jnp.zeros((n,), jnp.int32).at[perm].set(
        jnp.arange(n, dtype=jnp.int32))
    return out_s[:n][inv]


# speedup vs baseline: 1.6342x; 1.0378x over previous
import functools

import jax
import jax.numpy as jnp
from jax.experimental import pallas as pl

B = 512


def _w_block(pos_c, sq_col, geomT_ref, r2_val, a, b):
    sq_row = geomT_ref[2:3, b * B:(b + 1) * B]
    crossm2 = jnp.dot(pos_c, geomT_ref[0:2, b * B:(b + 1) * B],
                      preferred_element_type=jnp.float32)
    d2 = jnp.maximum((sq_col + sq_row) + crossm2, 1e-12)
    row_ids = a * B + jax.lax.broadcasted_iota(jnp.int32, (B, B), 0)
    col_ids = b * B + jax.lax.broadcasted_iota(jnp.int32, (B, B), 1)
    mask = (d2 <= r2_val) & (row_ids != col_ids)
    return jnp.where(mask, jax.lax.rsqrt(d2), 0.0)


def _sq_col(pos_c):
    px = pos_c[:, 0:1]
    py = pos_c[:, 1:2]
    return px * px + py * py


def _deg_kernel(nb, geomT_ref, pos_c_ref, x_ref, W1_ref, r2_ref,
                dinv_ref, M1_ref):
    a = pl.program_id(0)
    r2 = r2_ref[0:1, 0:1]
    pos_c = pos_c_ref[...]
    sq_col = _sq_col(pos_c)
    acc = jnp.zeros((B, 1), jnp.float32)
    for b in range(nb):
        w = _w_block(pos_c, sq_col, geomT_ref, r2, a, b)
        acc = acc + jnp.sum(w, axis=1, keepdims=True)
    deg = acc + 1.0
    dinv = jax.lax.rsqrt(deg)
    dinv_ref[...] = jnp.broadcast_to(dinv, (B, 8))
    xw = jnp.dot(x_ref[...], W1_ref[...], preferred_element_type=jnp.float32)
    M1_ref[...] = dinv * xw


def _agg_kernel(nb, relu_next, geomT_ref, pos_c_ref, M_ref, dinv_ref, Wn_ref,
                bias_ref, r2_ref, out_ref):
    a = pl.program_id(0)
    r2 = r2_ref[0:1, 0:1]
    pos_c = pos_c_ref[...]
    sq_col = _sq_col(pos_c)
    y = M_ref[pl.ds(a * B, B), :]
    for b in range(nb):
        w = _w_block(pos_c, sq_col, geomT_ref, r2, a, b)
        y = y + jnp.dot(w, M_ref[b * B:(b + 1) * B, :],
                        preferred_element_type=jnp.float32)
    dinv = dinv_ref[:, 0:1]
    y = dinv * y + bias_ref[0:1, :]
    if relu_next:
        h = jax.nn.relu(y)
        out_ref[...] = dinv * jnp.dot(h, Wn_ref[...],
                                      preferred_element_type=jnp.float32)
    else:
        out_ref[...] = y


def kernel(x, pos, r, W1, b1, W2, b2):
    n, feat = x.shape
    h1 = W1.shape[1]
    h2 = W2.shape[1]
    nb = -(-n // B)
    np_ = nb * B

    pad = np_ - n
    fill = 1e6 + 1e3 * jnp.arange(pad, dtype=jnp.float32)
    pos_p = jnp.concatenate([pos, jnp.stack([fill, fill], axis=1)], axis=0)
    x_p = jnp.concatenate([x, jnp.zeros((pad, feat), x.dtype)], axis=0)
    sq_p = jnp.sum(pos_p * pos_p, axis=1)
    geomT = jnp.concatenate([-2.0 * pos_p.T, sq_p[None, :],
                             jnp.zeros((5, np_), jnp.float32)], axis=0)
    r_f = jnp.asarray(r, jnp.float32)
    r2_b = jnp.full((1, 128), r_f * r_f, jnp.float32)
    b1_2 = b1.reshape(1, h1)
    b2_2 = b2.reshape(1, h2)

    full = lambda shape: pl.BlockSpec(shape, lambda a: (0, 0))
    rowblk = lambda w: pl.BlockSpec((B, w), lambda a: (a, 0))

    dinv, M1 = pl.pallas_call(
        functools.partial(_deg_kernel, nb),
        grid=(nb,),
        in_specs=[full((8, np_)), rowblk(2), rowblk(feat), full((feat, h1)),
                  full((1, 128))],
        out_specs=[rowblk(8), rowblk(h1)],
        out_shape=[jax.ShapeDtypeStruct((np_, 8), jnp.float32),
                   jax.ShapeDtypeStruct((np_, h1), jnp.float32)],
    )(geomT, pos_p, x_p, W1, r2_b)

    M2 = pl.pallas_call(
        functools.partial(_agg_kernel, nb, True),
        grid=(nb,),
        in_specs=[full((8, np_)), rowblk(2), full((np_, h1)), rowblk(8),
                  full((h1, h2)), full((1, h1)), full((1, 128))],
        out_specs=rowblk(h2),
        out_shape=jax.ShapeDtypeStruct((np_, h2), jnp.float32),
    )(geomT, pos_p, M1, dinv, W2, b1_2, r2_b)

    out = pl.pallas_call(
        functools.partial(_agg_kernel, nb, False),
        grid=(nb,),
        in_specs=[full((8, np_)), rowblk(2), full((np_, h2)), rowblk(8),
                  full((h1, h2)), full((1, h2)), full((1, 128))],
        out_specs=rowblk(h2),
        out_shape=jax.ShapeDtypeStruct((np_, h2), jnp.float32),
    )(geomT, pos_p, M2, dinv, W2, b2_2, r2_b)

    return out[:n]
